# Initial kernel scaffold; baseline (speedup 1.0000x reference)
import jax
import jax.numpy as jnp
from jax.experimental import pallas as pl


def kernel(x, edge_index, W1, b1, W2, b2, Wl, bl):
    raise NotImplementedError("write your pallas kernel here")


# speedup vs baseline: 4.4149x; 4.4149x over previous
import functools

import jax
import jax.numpy as jnp
from jax import lax
from jax.experimental import pallas as pl
from jax.experimental.pallas import tpu as pltpu
from jax.experimental.pallas import tpu_sc as plsc

N = 10000
E = 320000
H = 128
NCLS = 10
NC = 2
NS = 16
NW = NC * NS
CH = 128
E2 = 327680
EPW = E2 // NW
CPW = EPW // CH
ER = E2 // CH
N1 = 10112
RPT = N1 // NS

f32 = jnp.float32
i32 = jnp.int32


def _mesh():
    return plsc.VectorSubcoreMesh(
        core_axis_name="c", subcore_axis_name="s", num_cores=NC, num_subcores=NS
    )


@functools.cache
def _build_edge_prep():
    return pl.kernel(
        _edge_prep_body,
        out_type=[
            jax.ShapeDtypeStruct((ER, CH), i32),
            jax.ShapeDtypeStruct((2 * N1, H), f32),
        ],
        mesh=_mesh(),
        scratch_types=[
            pltpu.VMEM((CPW, CH), i32),
            pltpu.VMEM((CPW, CH), i32),
            pltpu.VMEM((1, CH), i32),
            pltpu.VMEM((CPW, CH), i32),
            pltpu.VMEM((CH, H), f32),
            pltpu.VMEM_SHARED((N1, H), f32),
        ],
    )


def _edge_prep_body(src_hbm, dst_hbm, zrow_hbm, ones_hbm, dstp_hbm, deg_hbm,
                    src_v, dst_v, srcp_v, dstp_v, ones_v, acc):
    cid = lax.axis_index("c")
    sid = lax.axis_index("s")
    wid = sid * NC + cid
    base = wid * CPW
    pltpu.sync_copy(zrow_hbm.at[pl.ds(sid * RPT, RPT)],
                    acc.at[pl.ds(sid * RPT, RPT)])
    pltpu.sync_copy(ones_hbm, ones_v)
    pltpu.sync_copy(src_hbm.at[pl.ds(base, CPW)], src_v)
    pltpu.sync_copy(dst_hbm.at[pl.ds(base, CPW)], dst_v)
    plsc.subcore_barrier()

    def chunk(c, carry):
        for j in range(CH // 16):
            s16 = src_v[c, pl.ds(j * 16, 16)]
            d16 = dst_v[c, pl.ds(j * 16, 16)]
            loop = s16 == d16
            srcp_v[0, pl.ds(j * 16, 16)] = jnp.where(loop, N, s16)
            dstp_v[c, pl.ds(j * 16, 16)] = jnp.where(loop, N, d16)
        pltpu.sync_copy(ones_v, acc.at[srcp_v.at[0]], add=True)
        return carry

    lax.fori_loop(0, CPW, chunk, 0)
    pltpu.sync_copy(dstp_v, dstp_hbm.at[pl.ds(base, CPW)])
    plsc.subcore_barrier()
    pltpu.sync_copy(acc.at[pl.ds(sid * RPT, RPT)],
                    deg_hbm.at[pl.ds(cid * N1 + sid * RPT, RPT)])


@functools.cache
def _build_prop():
    return pl.kernel(
        _prop_body,
        out_type=jax.ShapeDtypeStruct((2 * N1, H), f32),
        mesh=_mesh(),
        scratch_types=[
            pltpu.VMEM((CPW, CH), i32),
            pltpu.VMEM((CPW, CH), i32),
            pltpu.VMEM((CH, H), f32),
            pltpu.VMEM_SHARED((N1, H), f32),
            pltpu.SemaphoreType.DMA,
        ],
    )


def _prop_body(xs_hbm, src_hbm, dstp_hbm, zrow_hbm, out_hbm,
               src_v, dstp_v, rows_v, acc, sem):
    cid = lax.axis_index("c")
    sid = lax.axis_index("s")
    wid = sid * NC + cid
    base = wid * CPW
    pltpu.sync_copy(zrow_hbm.at[pl.ds(sid * RPT, RPT)],
                    acc.at[pl.ds(sid * RPT, RPT)])
    pltpu.sync_copy(src_hbm.at[pl.ds(base, CPW)], src_v)
    pltpu.sync_copy(dstp_hbm.at[pl.ds(base, CPW)], dstp_v)
    plsc.subcore_barrier()

    def chunk(c, carry):
        pltpu.async_copy(xs_hbm.at[src_v.at[c]], rows_v, sem).wait()
        pltpu.sync_copy(rows_v, acc.at[dstp_v.at[c]], add=True)
        return carry

    lax.fori_loop(0, CPW, chunk, 0)
    plsc.subcore_barrier()
    pltpu.sync_copy(acc.at[pl.ds(sid * RPT, RPT)],
                    out_hbm.at[pl.ds(cid * N1 + sid * RPT, RPT)])


BN = 1000
GRID = (N // BN,)


def _row_spec():
    return pl.BlockSpec((BN, H), lambda i: (i, 0))


def _dis_spec():
    return pl.BlockSpec((BN, 16), lambda i: (i, 0))


def _w_spec():
    return pl.BlockSpec((H, H), lambda i: (0, 0))


def _b_spec():
    return pl.BlockSpec((1, H), lambda i: (0, 0))


def _dot(a, b):
    return jnp.dot(a, b, preferred_element_type=f32)


def _stage_b_body(dega_ref, degb_ref, x_ref, dis_ref, xs_ref):
    deg = dega_ref[...][:, :1] + degb_ref[...][:, :1]
    dis = jnp.where(deg > 0.0, 1.0 / jnp.sqrt(jnp.maximum(deg, 1e-12)), 0.0)
    dis_ref[...] = jnp.broadcast_to(dis, dis_ref.shape)
    xs_ref[...] = dis * x_ref[...]


def _stage_c_body(pa_ref, pb_ref, dis_ref, x_ref, w10_ref, w11_ref,
                  u1_ref, acc_ref):
    d = dis_ref[...][:, :1]
    tx1 = -(d * (pa_ref[...] + pb_ref[...]))
    u1_ref[...] = d * tx1
    acc_ref[...] = _dot(x_ref[...], w10_ref[...]) + _dot(tx1, w11_ref[...])


def _stage_d_body(pa_ref, pb_ref, dis_ref, x_ref, acc_ref, w12_ref, b1_ref,
                  h_ref, hs_ref):
    d = dis_ref[...][:, :1]
    tx2 = -2.0 * (d * (pa_ref[...] + pb_ref[...])) - x_ref[...]
    pre = acc_ref[...] + _dot(tx2, w12_ref[...]) + b1_ref[...]
    h = jnp.maximum(pre, 0.0)
    h_ref[...] = h
    hs_ref[...] = d * h


def _stage_e_body(pa_ref, pb_ref, dis_ref, h_ref, w20_ref, w21_ref,
                  u2_ref, acc2_ref):
    d = dis_ref[...][:, :1]
    t1 = -(d * (pa_ref[...] + pb_ref[...]))
    u2_ref[...] = d * t1
    acc2_ref[...] = _dot(h_ref[...], w20_ref[...]) + _dot(t1, w21_ref[...])


def _stage_f_body(pa_ref, pb_ref, dis_ref, h_ref, acc2_ref, w22_ref, b2_ref,
                  wl_ref, bl_ref, o_ref):
    d = dis_ref[...][:, :1]
    t2 = -2.0 * (d * (pa_ref[...] + pb_ref[...])) - h_ref[...]
    hf = h_ref[...] + acc2_ref[...] + _dot(t2, w22_ref[...]) + b2_ref[...]
    logits = _dot(hf, wl_ref[...]) + bl_ref[...]
    lane = lax.broadcasted_iota(i32, logits.shape, 1)
    valid = lane < NCLS
    masked = jnp.where(valid, logits, -1e30)
    m = jnp.max(masked, axis=1, keepdims=True)
    ex = jnp.where(valid, jnp.exp(logits - m), 0.0)
    lse = jnp.log(jnp.sum(ex, axis=1, keepdims=True)) + m
    o_ref[...] = logits - lse


_stage_b = pl.pallas_call(
    _stage_b_body,
    grid=GRID,
    in_specs=[_row_spec(), _row_spec(), _row_spec()],
    out_specs=[_dis_spec(), _row_spec()],
    out_shape=[jax.ShapeDtypeStruct((N, 16), f32),
               jax.ShapeDtypeStruct((N, H), f32)],
)

_stage_c = pl.pallas_call(
    _stage_c_body,
    grid=GRID,
    in_specs=[_row_spec(), _row_spec(), _dis_spec(), _row_spec(),
              _w_spec(), _w_spec()],
    out_specs=[_row_spec(), _row_spec()],
    out_shape=[jax.ShapeDtypeStruct((N, H), f32),
               jax.ShapeDtypeStruct((N, H), f32)],
)

_stage_d = pl.pallas_call(
    _stage_d_body,
    grid=GRID,
    in_specs=[_row_spec(), _row_spec(), _dis_spec(), _row_spec(), _row_spec(),
              _w_spec(), _b_spec()],
    out_specs=[_row_spec(), _row_spec()],
    out_shape=[jax.ShapeDtypeStruct((N, H), f32),
               jax.ShapeDtypeStruct((N, H), f32)],
)

_stage_e = pl.pallas_call(
    _stage_e_body,
    grid=GRID,
    in_specs=[_row_spec(), _row_spec(), _dis_spec(), _row_spec(),
              _w_spec(), _w_spec()],
    out_specs=[_row_spec(), _row_spec()],
    out_shape=[jax.ShapeDtypeStruct((N, H), f32),
               jax.ShapeDtypeStruct((N, H), f32)],
)

_stage_f = pl.pallas_call(
    _stage_f_body,
    grid=GRID,
    in_specs=[_row_spec(), _row_spec(), _dis_spec(), _row_spec(), _row_spec(),
              _w_spec(), _b_spec(), _w_spec(), _b_spec()],
    out_specs=_row_spec(),
    out_shape=jax.ShapeDtypeStruct((N, H), f32),
)


def kernel(x, edge_index, W1, b1, W2, b2, Wl, bl):
    pad = jnp.zeros((E2 - E,), i32)
    src = jnp.concatenate([edge_index[0].astype(i32), pad]).reshape(ER, CH)
    dst = jnp.concatenate([edge_index[1].astype(i32), pad]).reshape(ER, CH)
    ones = jnp.ones((CH, H), f32)
    zrow = jnp.zeros((N1, H), f32)

    _edge_prep = _build_edge_prep()
    _prop = _build_prop()
    dstp, degp = _edge_prep(src, dst, zrow, ones)
    dis16, xs = _stage_b(degp[0:N], degp[N1:N1 + N], x)

    p = _prop(xs, src, dstp, zrow)
    u1, acc1 = _stage_c(p[0:N], p[N1:N1 + N], dis16, x, W1[0], W1[1])

    p = _prop(u1, src, dstp, zrow)
    h, hs = _stage_d(p[0:N], p[N1:N1 + N], dis16, x, acc1, W1[2],
                     b1.reshape(1, H))

    p = _prop(hs, src, dstp, zrow)
    u2, acc2 = _stage_e(p[0:N], p[N1:N1 + N], dis16, h, W2[0], W2[1])

    p = _prop(u2, src, dstp, zrow)
    wlp = jnp.zeros((H, H), f32).at[:, :NCLS].set(Wl)
    blp = jnp.zeros((1, H), f32).at[0, :NCLS].set(bl)
    outp = _stage_f(p[0:N], p[N1:N1 + N], dis16, h, acc2, W2[2],
                    b2.reshape(1, H), wlp, blp)

    return outp[:, :NCLS], edge_index


# speedup vs baseline: 4.6930x; 1.0630x over previous
import functools

import jax
import jax.numpy as jnp
from jax import lax
from jax.experimental import pallas as pl
from jax.experimental.pallas import tpu as pltpu
from jax.experimental.pallas import tpu_sc as plsc

N = 10000
E = 320000
H = 128
NCLS = 10
NC = 2
NS = 16
NW = NC * NS
CH = 128
E2 = 327680
EPW = E2 // NW
CPW = EPW // CH
ER = E2 // CH
N1 = 10112
RPT = N1 // NS

f32 = jnp.float32
i32 = jnp.int32


def _mesh():
    return plsc.VectorSubcoreMesh(
        core_axis_name="c", subcore_axis_name="s", num_cores=NC, num_subcores=NS
    )


@functools.cache
def _build_edge_prep():
    return pl.kernel(
        _edge_prep_body,
        out_type=[
            jax.ShapeDtypeStruct((E2,), i32),
            jax.ShapeDtypeStruct((2 * N1, H), f32),
        ],
        mesh=_mesh(),
        scratch_types=[
            pltpu.VMEM((EPW,), i32),
            pltpu.VMEM((EPW,), i32),
            pltpu.VMEM((CH,), i32),
            pltpu.VMEM((EPW,), i32),
            pltpu.VMEM((CH, H), f32),
            pltpu.VMEM_SHARED((N1, H), f32),
        ],
    )


def _edge_prep_body(src_hbm, dst_hbm, zrow_hbm, ones_hbm, dstp_hbm, deg_hbm,
                    src_v, dst_v, srcp_c, dstp_v, ones_v, acc):
    cid = lax.axis_index("c")
    sid = lax.axis_index("s")
    wid = sid * NC + cid
    base = wid * EPW
    pltpu.sync_copy(zrow_hbm.at[pl.ds(sid * RPT, RPT)],
                    acc.at[pl.ds(sid * RPT, RPT)])
    pltpu.sync_copy(ones_hbm, ones_v)
    pltpu.sync_copy(src_hbm.at[pl.ds(base, EPW)], src_v)
    pltpu.sync_copy(dst_hbm.at[pl.ds(base, EPW)], dst_v)
    plsc.subcore_barrier()

    def chunk(c, carry):
        for j in range(CH // 16):
            s16 = src_v[pl.ds(c * CH + j * 16, 16)]
            d16 = dst_v[pl.ds(c * CH + j * 16, 16)]
            loop = s16 == d16
            srcp_c[pl.ds(j * 16, 16)] = jnp.where(loop, N, s16)
            dstp_v[pl.ds(c * CH + j * 16, 16)] = jnp.where(loop, N, d16)
        pltpu.sync_copy(ones_v, acc.at[srcp_c], add=True)
        return carry

    lax.fori_loop(0, CPW, chunk, 0)
    pltpu.sync_copy(dstp_v, dstp_hbm.at[pl.ds(base, EPW)])
    plsc.subcore_barrier()
    pltpu.sync_copy(acc.at[pl.ds(sid * RPT, RPT)],
                    deg_hbm.at[pl.ds(cid * N1 + sid * RPT, RPT)])


@functools.cache
def _build_prop():
    return pl.kernel(
        _prop_body,
        out_type=jax.ShapeDtypeStruct((2 * N1, H), f32),
        mesh=_mesh(),
        scratch_types=[
            pltpu.VMEM((EPW,), i32),
            pltpu.VMEM((CH,), i32),
            pltpu.VMEM((CH,), i32),
            pltpu.VMEM((CH, H), f32),
            pltpu.VMEM((CH, H), f32),
            pltpu.VMEM_SHARED((N1, H), f32),
            pltpu.SemaphoreType.DMA,
        ],
    )


def _prop_body(xs_hbm, src_hbm, dstp_hbm, zrow_hbm, out_hbm,
               src_v, dstp_a, dstp_b, rows_a, rows_b, acc, gsem):
    cid = lax.axis_index("c")
    sid = lax.axis_index("s")
    wid = sid * NC + cid
    base = wid * EPW
    pltpu.sync_copy(zrow_hbm.at[pl.ds(sid * RPT, RPT)],
                    acc.at[pl.ds(sid * RPT, RPT)])
    pltpu.sync_copy(src_hbm.at[pl.ds(base, EPW)], src_v)
    plsc.subcore_barrier()

    def gather(coff, rows):
        return pltpu.make_async_copy(
            xs_hbm.at[src_v.at[pl.ds(coff, CH)]], rows, gsem)

    pltpu.sync_copy(dstp_hbm.at[pl.ds(base, CH)], dstp_a)
    gather(0, rows_a).start()

    def pair(i, carry):
        c1 = (2 * i + 1) * CH
        nxt = 2 * i + 2
        c2 = jnp.where(nxt >= CPW, 0, nxt) * CH
        pltpu.sync_copy(dstp_hbm.at[pl.ds(base + c1, CH)], dstp_b)
        gather(0, rows_a).wait()
        gather(c1, rows_b).start()
        pltpu.sync_copy(rows_a, acc.at[dstp_a], add=True)
        pltpu.sync_copy(dstp_hbm.at[pl.ds(base + c2, CH)], dstp_a)
        gather(0, rows_b).wait()
        gather(c2, rows_a).start()
        pltpu.sync_copy(rows_b, acc.at[dstp_b], add=True)
        return carry

    lax.fori_loop(0, CPW // 2, pair, 0)
    gather(0, rows_a).wait()
    plsc.subcore_barrier()
    pltpu.sync_copy(acc.at[pl.ds(sid * RPT, RPT)],
                    out_hbm.at[pl.ds(cid * N1 + sid * RPT, RPT)])


BN = 1000
GRID = (N // BN,)


def _row_spec():
    return pl.BlockSpec((BN, H), lambda i: (i, 0))


def _dis_spec():
    return pl.BlockSpec((BN, 16), lambda i: (i, 0))


def _w_spec():
    return pl.BlockSpec((H, H), lambda i: (0, 0))


def _b_spec():
    return pl.BlockSpec((1, H), lambda i: (0, 0))


def _dot(a, b):
    return jnp.dot(a, b, preferred_element_type=f32)


def _stage_b_body(dega_ref, degb_ref, x_ref, dis_ref, xs_ref):
    deg = dega_ref[...][:, :1] + degb_ref[...][:, :1]
    dis = jnp.where(deg > 0.0, 1.0 / jnp.sqrt(jnp.maximum(deg, 1e-12)), 0.0)
    dis_ref[...] = jnp.broadcast_to(dis, dis_ref.shape)
    xs_ref[...] = dis * x_ref[...]


def _stage_c_body(pa_ref, pb_ref, dis_ref, x_ref, w10_ref, w11_ref,
                  u1_ref, acc_ref):
    d = dis_ref[...][:, :1]
    tx1 = -(d * (pa_ref[...] + pb_ref[...]))
    u1_ref[...] = d * tx1
    acc_ref[...] = _dot(x_ref[...], w10_ref[...]) + _dot(tx1, w11_ref[...])


def _stage_d_body(pa_ref, pb_ref, dis_ref, x_ref, acc_ref, w12_ref, b1_ref,
                  h_ref, hs_ref):
    d = dis_ref[...][:, :1]
    tx2 = -2.0 * (d * (pa_ref[...] + pb_ref[...])) - x_ref[...]
    pre = acc_ref[...] + _dot(tx2, w12_ref[...]) + b1_ref[...]
    h = jnp.maximum(pre, 0.0)
    h_ref[...] = h
    hs_ref[...] = d * h


def _stage_e_body(pa_ref, pb_ref, dis_ref, h_ref, w20_ref, w21_ref,
                  u2_ref, acc2_ref):
    d = dis_ref[...][:, :1]
    t1 = -(d * (pa_ref[...] + pb_ref[...]))
    u2_ref[...] = d * t1
    acc2_ref[...] = _dot(h_ref[...], w20_ref[...]) + _dot(t1, w21_ref[...])


def _stage_f_body(pa_ref, pb_ref, dis_ref, h_ref, acc2_ref, w22_ref, b2_ref,
                  wl_ref, bl_ref, o_ref):
    d = dis_ref[...][:, :1]
    t2 = -2.0 * (d * (pa_ref[...] + pb_ref[...])) - h_ref[...]
    hf = h_ref[...] + acc2_ref[...] + _dot(t2, w22_ref[...]) + b2_ref[...]
    logits = _dot(hf, wl_ref[...]) + bl_ref[...]
    lane = lax.broadcasted_iota(i32, logits.shape, 1)
    valid = lane < NCLS
    masked = jnp.where(valid, logits, -1e30)
    m = jnp.max(masked, axis=1, keepdims=True)
    ex = jnp.where(valid, jnp.exp(logits - m), 0.0)
    lse = jnp.log(jnp.sum(ex, axis=1, keepdims=True)) + m
    o_ref[...] = logits - lse


_stage_b = pl.pallas_call(
    _stage_b_body,
    grid=GRID,
    in_specs=[_row_spec(), _row_spec(), _row_spec()],
    out_specs=[_dis_spec(), _row_spec()],
    out_shape=[jax.ShapeDtypeStruct((N, 16), f32),
               jax.ShapeDtypeStruct((N, H), f32)],
)

_stage_c = pl.pallas_call(
    _stage_c_body,
    grid=GRID,
    in_specs=[_row_spec(), _row_spec(), _dis_spec(), _row_spec(),
              _w_spec(), _w_spec()],
    out_specs=[_row_spec(), _row_spec()],
    out_shape=[jax.ShapeDtypeStruct((N, H), f32),
               jax.ShapeDtypeStruct((N, H), f32)],
)

_stage_d = pl.pallas_call(
    _stage_d_body,
    grid=GRID,
    in_specs=[_row_spec(), _row_spec(), _dis_spec(), _row_spec(), _row_spec(),
              _w_spec(), _b_spec()],
    out_specs=[_row_spec(), _row_spec()],
    out_shape=[jax.ShapeDtypeStruct((N, H), f32),
               jax.ShapeDtypeStruct((N, H), f32)],
)

_stage_e = pl.pallas_call(
    _stage_e_body,
    grid=GRID,
    in_specs=[_row_spec(), _row_spec(), _dis_spec(), _row_spec(),
              _w_spec(), _w_spec()],
    out_specs=[_row_spec(), _row_spec()],
    out_shape=[jax.ShapeDtypeStruct((N, H), f32),
               jax.ShapeDtypeStruct((N, H), f32)],
)

_stage_f = pl.pallas_call(
    _stage_f_body,
    grid=GRID,
    in_specs=[_row_spec(), _row_spec(), _dis_spec(), _row_spec(), _row_spec(),
              _w_spec(), _b_spec(), _w_spec(), _b_spec()],
    out_specs=_row_spec(),
    out_shape=jax.ShapeDtypeStruct((N, H), f32),
)


def kernel(x, edge_index, W1, b1, W2, b2, Wl, bl):
    pad = jnp.zeros((E2 - E,), i32)
    src = jnp.concatenate([edge_index[0].astype(i32), pad])
    dst = jnp.concatenate([edge_index[1].astype(i32), pad])
    ones = jnp.ones((CH, H), f32)
    zrow = jnp.zeros((N1, H), f32)

    _edge_prep = _build_edge_prep()
    _prop = _build_prop()
    dstp, degp = _edge_prep(src, dst, zrow, ones)
    dis16, xs = _stage_b(degp[0:N], degp[N1:N1 + N], x)

    p = _prop(xs, src, dstp, zrow)
    u1, acc1 = _stage_c(p[0:N], p[N1:N1 + N], dis16, x, W1[0], W1[1])

    p = _prop(u1, src, dstp, zrow)
    h, hs = _stage_d(p[0:N], p[N1:N1 + N], dis16, x, acc1, W1[2],
                     b1.reshape(1, H))

    p = _prop(hs, src, dstp, zrow)
    u2, acc2 = _stage_e(p[0:N], p[N1:N1 + N], dis16, h, W2[0], W2[1])

    p = _prop(u2, src, dstp, zrow)
    wlp = jnp.zeros((H, H), f32).at[:, :NCLS].set(Wl)
    blp = jnp.zeros((1, H), f32).at[0, :NCLS].set(bl)
    outp = _stage_f(p[0:N], p[N1:N1 + N], dis16, h, acc2, W2[2],
                    b2.reshape(1, H), wlp, blp)

    return outp[:, :NCLS], edge_index


# speedup vs baseline: 5.5816x; 1.1893x over previous
import functools

import jax
import jax.numpy as jnp
from jax import lax
from jax.experimental import pallas as pl
from jax.experimental.pallas import tpu as pltpu
from jax.experimental.pallas import tpu_sc as plsc

N = 10000
E = 320000
H = 128
NCLS = 10
NC = 2
NS = 16
NW = NC * NS
CH = 128
E2 = 327680
EPW = E2 // NW
CPW = EPW // CH
ER = E2 // CH
N1 = 10112
RPT = N1 // NS

f32 = jnp.float32
i32 = jnp.int32


def _mesh():
    return plsc.VectorSubcoreMesh(
        core_axis_name="c", subcore_axis_name="s", num_cores=NC, num_subcores=NS
    )


@functools.cache
def _build_edge_prep():
    return pl.kernel(
        _edge_prep_body,
        out_type=[
            jax.ShapeDtypeStruct((E2,), i32),
            jax.ShapeDtypeStruct((2 * N1, H), f32),
        ],
        mesh=_mesh(),
        scratch_types=[
            pltpu.VMEM((EPW,), i32),
            pltpu.VMEM((EPW,), i32),
            pltpu.VMEM((CH,), i32),
            pltpu.VMEM((EPW,), i32),
            pltpu.VMEM((CH, H), f32),
            pltpu.VMEM_SHARED((N1, H), f32),
        ],
    )


def _edge_prep_body(src_hbm, dst_hbm, zrow_hbm, ones_hbm, dstp_hbm, deg_hbm,
                    src_v, dst_v, srcp_c, dstp_v, ones_v, acc):
    cid = lax.axis_index("c")
    sid = lax.axis_index("s")
    wid = sid * NC + cid
    base = wid * EPW
    pltpu.sync_copy(zrow_hbm.at[pl.ds(sid * RPT, RPT)],
                    acc.at[pl.ds(sid * RPT, RPT)])
    pltpu.sync_copy(ones_hbm, ones_v)
    pltpu.sync_copy(src_hbm.at[pl.ds(base, EPW)], src_v)
    pltpu.sync_copy(dst_hbm.at[pl.ds(base, EPW)], dst_v)
    plsc.subcore_barrier()

    def chunk(c, carry):
        for j in range(CH // 16):
            s16 = src_v[pl.ds(c * CH + j * 16, 16)]
            d16 = dst_v[pl.ds(c * CH + j * 16, 16)]
            loop = s16 == d16
            srcp_c[pl.ds(j * 16, 16)] = jnp.where(loop, N, s16)
            dstp_v[pl.ds(c * CH + j * 16, 16)] = jnp.where(loop, N, d16)
        pltpu.sync_copy(ones_v, acc.at[srcp_c], add=True)
        return carry

    lax.fori_loop(0, CPW, chunk, 0)
    pltpu.sync_copy(dstp_v, dstp_hbm.at[pl.ds(base, EPW)])
    plsc.subcore_barrier()
    pltpu.sync_copy(acc.at[pl.ds(sid * RPT, RPT)],
                    deg_hbm.at[pl.ds(cid * N1 + sid * RPT, RPT)])


P0 = 126
P1 = (2 * CPW) - P0
E3 = E2 + (P0 - P1) * CH


@functools.cache
def _build_prop():
    return pl.kernel(
        _prop_body,
        out_type=jax.ShapeDtypeStruct((2 * N1, H), f32),
        mesh=_mesh(),
        scratch_types=[
            pltpu.VMEM((P0 * CH,), i32),
            pltpu.VMEM((CH,), i32),
            pltpu.VMEM((CH,), i32),
            pltpu.VMEM((CH, H), f32),
            pltpu.VMEM((CH, H), f32),
            pltpu.VMEM_SHARED((N1, H), f32),
            pltpu.SemaphoreType.DMA,
        ],
    )


def _prop_body(xs_hbm, src_hbm, dstp_hbm, zrow_hbm, out_hbm,
               src_v, dstp_a, dstp_b, rows_a, rows_b, acc, gsem):
    cid = lax.axis_index("c")
    sid = lax.axis_index("s")
    cpw = jnp.where(cid == 0, P0, P1)
    base = sid * (P0 + P1) * CH + cid * (P0 * CH)
    pltpu.sync_copy(zrow_hbm.at[pl.ds(sid * RPT, RPT)],
                    acc.at[pl.ds(sid * RPT, RPT)])
    pltpu.sync_copy(src_hbm.at[pl.ds(base, P0 * CH)], src_v)
    plsc.subcore_barrier()

    def gather(coff, rows):
        return pltpu.make_async_copy(
            xs_hbm.at[src_v.at[pl.ds(coff, CH)]], rows, gsem)

    pltpu.sync_copy(dstp_hbm.at[pl.ds(base, CH)], dstp_a)
    gather(0, rows_a).start()

    def pair(i, carry):
        c1 = (2 * i + 1) * CH
        nxt = 2 * i + 2
        c2 = jnp.where(nxt >= cpw, 0, nxt) * CH
        pltpu.sync_copy(dstp_hbm.at[pl.ds(base + c1, CH)], dstp_b)
        gather(0, rows_a).wait()
        gather(c1, rows_b).start()
        pltpu.sync_copy(rows_a, acc.at[dstp_a], add=True)
        pltpu.sync_copy(dstp_hbm.at[pl.ds(base + c2, CH)], dstp_a)
        gather(0, rows_b).wait()
        gather(c2, rows_a).start()
        pltpu.sync_copy(rows_b, acc.at[dstp_b], add=True)
        return carry

    lax.fori_loop(0, cpw // 2, pair, 0)
    gather(0, rows_a).wait()
    plsc.subcore_barrier()
    pltpu.sync_copy(acc.at[pl.ds(sid * RPT, RPT)],
                    out_hbm.at[pl.ds(cid * N1 + sid * RPT, RPT)])


BN = 1000
GRID = (N // BN,)


def _row_spec():
    return pl.BlockSpec((BN, H), lambda i: (i, 0))


def _dis_spec():
    return pl.BlockSpec((BN, 16), lambda i: (i, 0))


def _w_spec():
    return pl.BlockSpec((H, H), lambda i: (0, 0))


def _b_spec():
    return pl.BlockSpec((1, H), lambda i: (0, 0))


def _dot(a, b):
    return jnp.dot(a, b, preferred_element_type=f32)


def _stage_b_body(dega_ref, degb_ref, x_ref, dis_ref, xs_ref):
    deg = dega_ref[...][:, :1] + degb_ref[...][:, :1]
    dis = jnp.where(deg > 0.0, 1.0 / jnp.sqrt(jnp.maximum(deg, 1e-12)), 0.0)
    dis_ref[...] = jnp.broadcast_to(dis, dis_ref.shape)
    xs_ref[...] = dis * x_ref[...]


def _stage_c_body(pa_ref, pb_ref, dis_ref, x_ref, w10_ref, w11_ref,
                  u1_ref, acc_ref):
    d = dis_ref[...][:, :1]
    tx1 = -(d * (pa_ref[...] + pb_ref[...]))
    u1_ref[...] = d * tx1
    acc_ref[...] = _dot(x_ref[...], w10_ref[...]) + _dot(tx1, w11_ref[...])


def _stage_d_body(pa_ref, pb_ref, dis_ref, x_ref, acc_ref, w12_ref, b1_ref,
                  h_ref, hs_ref):
    d = dis_ref[...][:, :1]
    tx2 = -2.0 * (d * (pa_ref[...] + pb_ref[...])) - x_ref[...]
    pre = acc_ref[...] + _dot(tx2, w12_ref[...]) + b1_ref[...]
    h = jnp.maximum(pre, 0.0)
    h_ref[...] = h
    hs_ref[...] = d * h


def _stage_e_body(pa_ref, pb_ref, dis_ref, h_ref, w20_ref, w21_ref,
                  u2_ref, acc2_ref):
    d = dis_ref[...][:, :1]
    t1 = -(d * (pa_ref[...] + pb_ref[...]))
    u2_ref[...] = d * t1
    acc2_ref[...] = _dot(h_ref[...], w20_ref[...]) + _dot(t1, w21_ref[...])


def _stage_f_body(pa_ref, pb_ref, dis_ref, h_ref, acc2_ref, w22_ref, b2_ref,
                  wl_ref, bl_ref, o_ref):
    d = dis_ref[...][:, :1]
    t2 = -2.0 * (d * (pa_ref[...] + pb_ref[...])) - h_ref[...]
    hf = h_ref[...] + acc2_ref[...] + _dot(t2, w22_ref[...]) + b2_ref[...]
    logits = _dot(hf, wl_ref[...]) + bl_ref[...]
    lane = lax.broadcasted_iota(i32, logits.shape, 1)
    valid = lane < NCLS
    masked = jnp.where(valid, logits, -1e30)
    m = jnp.max(masked, axis=1, keepdims=True)
    ex = jnp.where(valid, jnp.exp(logits - m), 0.0)
    lse = jnp.log(jnp.sum(ex, axis=1, keepdims=True)) + m
    o_ref[...] = logits - lse


_stage_b = pl.pallas_call(
    _stage_b_body,
    grid=GRID,
    in_specs=[_row_spec(), _row_spec(), _row_spec()],
    out_specs=[_dis_spec(), _row_spec()],
    out_shape=[jax.ShapeDtypeStruct((N, 16), f32),
               jax.ShapeDtypeStruct((N, H), f32)],
)

_stage_c = pl.pallas_call(
    _stage_c_body,
    grid=GRID,
    in_specs=[_row_spec(), _row_spec(), _dis_spec(), _row_spec(),
              _w_spec(), _w_spec()],
    out_specs=[_row_spec(), _row_spec()],
    out_shape=[jax.ShapeDtypeStruct((N, H), f32),
               jax.ShapeDtypeStruct((N, H), f32)],
)

_stage_d = pl.pallas_call(
    _stage_d_body,
    grid=GRID,
    in_specs=[_row_spec(), _row_spec(), _dis_spec(), _row_spec(), _row_spec(),
              _w_spec(), _b_spec()],
    out_specs=[_row_spec(), _row_spec()],
    out_shape=[jax.ShapeDtypeStruct((N, H), f32),
               jax.ShapeDtypeStruct((N, H), f32)],
)

_stage_e = pl.pallas_call(
    _stage_e_body,
    grid=GRID,
    in_specs=[_row_spec(), _row_spec(), _dis_spec(), _row_spec(),
              _w_spec(), _w_spec()],
    out_specs=[_row_spec(), _row_spec()],
    out_shape=[jax.ShapeDtypeStruct((N, H), f32),
               jax.ShapeDtypeStruct((N, H), f32)],
)

_stage_f = pl.pallas_call(
    _stage_f_body,
    grid=GRID,
    in_specs=[_row_spec(), _row_spec(), _dis_spec(), _row_spec(), _row_spec(),
              _w_spec(), _b_spec(), _w_spec(), _b_spec()],
    out_specs=_row_spec(),
    out_shape=jax.ShapeDtypeStruct((N, H), f32),
)


def kernel(x, edge_index, W1, b1, W2, b2, Wl, bl):
    src = jnp.concatenate([edge_index[0].astype(i32), jnp.zeros((E3 - E,), i32)])
    dst = jnp.concatenate([edge_index[1].astype(i32), jnp.zeros((E2 - E,), i32)])
    ones = jnp.ones((CH, H), f32)
    zrow = jnp.zeros((N1, H), f32)

    _edge_prep = _build_edge_prep()
    _prop = _build_prop()
    dstp, degp = _edge_prep(src, dst, zrow, ones)
    dis16, xs = _stage_b(degp[0:N], degp[N1:N1 + N], x)

    p = _prop(xs, src, dstp, zrow)
    u1, acc1 = _stage_c(p[0:N], p[N1:N1 + N], dis16, x, W1[0], W1[1])

    p = _prop(u1, src, dstp, zrow)
    h, hs = _stage_d(p[0:N], p[N1:N1 + N], dis16, x, acc1, W1[2],
                     b1.reshape(1, H))

    p = _prop(hs, src, dstp, zrow)
    u2, acc2 = _stage_e(p[0:N], p[N1:N1 + N], dis16, h, W2[0], W2[1])

    p = _prop(u2, src, dstp, zrow)
    wlp = jnp.zeros((H, H), f32).at[:, :NCLS].set(Wl)
    blp = jnp.zeros((1, H), f32).at[0, :NCLS].set(bl)
    outp = _stage_f(p[0:N], p[N1:N1 + N], dis16, h, acc2, W2[2],
                    b2.reshape(1, H), wlp, blp)

    return outp[:, :NCLS], edge_index


# speedup vs baseline: 6.0873x; 1.0906x over previous
import functools

import jax
import jax.numpy as jnp
from jax import lax
from jax.experimental import pallas as pl
from jax.experimental.pallas import tpu as pltpu
from jax.experimental.pallas import tpu_sc as plsc

N = 10000
E = 320000
H = 128
NCLS = 10
NC = 2
NS = 16
NW = NC * NS
CH = 128
E2 = 327680
EPW = E2 // NW
CPW = EPW // CH
ER = E2 // CH
N1 = 10112
RPT = N1 // NS

f32 = jnp.float32
i32 = jnp.int32


def _mesh():
    return plsc.VectorSubcoreMesh(
        core_axis_name="c", subcore_axis_name="s", num_cores=NC, num_subcores=NS
    )


@functools.cache
def _build_edge_prep():
    return pl.kernel(
        _edge_prep_body,
        out_type=[
            jax.ShapeDtypeStruct((E2,), i32),
            jax.ShapeDtypeStruct((2 * N1, H), f32),
        ],
        mesh=_mesh(),
        scratch_types=[
            pltpu.VMEM((EPW,), i32),
            pltpu.VMEM((EPW,), i32),
            pltpu.VMEM((CH,), i32),
            pltpu.VMEM((EPW,), i32),
            pltpu.VMEM((CH, H), f32),
            pltpu.VMEM_SHARED((N1, H), f32),
        ],
    )


def _edge_prep_body(src_hbm, dst_hbm, zrow_hbm, ones_hbm, dstp_hbm, deg_hbm,
                    src_v, dst_v, srcp_c, dstp_v, ones_v, acc):
    cid = lax.axis_index("c")
    sid = lax.axis_index("s")
    wid = sid * NC + cid
    base = wid * EPW
    pltpu.sync_copy(zrow_hbm.at[pl.ds(sid * RPT, RPT)],
                    acc.at[pl.ds(sid * RPT, RPT)])
    pltpu.sync_copy(ones_hbm, ones_v)
    pltpu.sync_copy(src_hbm.at[pl.ds(base, EPW)], src_v)
    pltpu.sync_copy(dst_hbm.at[pl.ds(base, EPW)], dst_v)
    plsc.subcore_barrier()

    def chunk(c, carry):
        for j in range(CH // 16):
            s16 = src_v[pl.ds(c * CH + j * 16, 16)]
            d16 = dst_v[pl.ds(c * CH + j * 16, 16)]
            loop = s16 == d16
            srcp_c[pl.ds(j * 16, 16)] = jnp.where(loop, N, s16)
            dstp_v[pl.ds(c * CH + j * 16, 16)] = jnp.where(loop, N, d16)
        pltpu.sync_copy(ones_v, acc.at[srcp_c], add=True)
        return carry

    lax.fori_loop(0, CPW, chunk, 0)
    pltpu.sync_copy(dstp_v, dstp_hbm.at[pl.ds(base, EPW)])
    plsc.subcore_barrier()
    pltpu.sync_copy(acc.at[pl.ds(sid * RPT, RPT)],
                    deg_hbm.at[pl.ds(cid * N1 + sid * RPT, RPT)])


CH2 = 64
TPC = E2 // (NS * CH2)
P0 = 240
P1 = TPC - P0
RNG = 4
QN = 8


@functools.cache
def _build_prop():
    return pl.kernel(
        _prop_body,
        out_type=jax.ShapeDtypeStruct((2 * N1, H), f32),
        mesh=_mesh(),
        scratch_types=[
            pltpu.VMEM((QN, CH2), i32),
            pltpu.VMEM((QN, CH2), i32),
            pltpu.VMEM((RNG, CH2, H), f32),
            pltpu.VMEM_SHARED((N1, H), f32),
            pltpu.SemaphoreType.DMA,
            pltpu.SemaphoreType.DMA,
            pltpu.SemaphoreType.DMA,
        ],
    )


def _prop_body(xs_hbm, src_hbm, dstp_hbm, zrow_hbm, out_hbm,
               sidx, didx, rows, acc, gsem, ssem, isem):
    cid = lax.axis_index("c")
    sid = lax.axis_index("s")
    cpw = jnp.where(cid == 0, P0, P1)
    base = sid * (TPC * CH2) + cid * (P0 * CH2)
    pltpu.sync_copy(zrow_hbm.at[pl.ds(sid * RPT, RPT)],
                    acc.at[pl.ds(sid * RPT, RPT)])
    plsc.subcore_barrier()

    def ioff(c):
        cw = jnp.where(c < cpw, c, 0)
        return base + cw * CH2

    def idx_start(c, q):
        pltpu.make_async_copy(
            src_hbm.at[pl.ds(ioff(c), CH2)], sidx.at[q], isem).start()
        pltpu.make_async_copy(
            dstp_hbm.at[pl.ds(ioff(c), CH2)], didx.at[q], isem).start()

    def idx_wait(q):
        pltpu.make_async_copy(
            src_hbm.at[pl.ds(base, CH2)], sidx.at[q], isem).wait()
        pltpu.make_async_copy(
            dstp_hbm.at[pl.ds(base, CH2)], didx.at[q], isem).wait()

    def g_desc(q, r):
        return pltpu.make_async_copy(xs_hbm.at[sidx.at[q]], rows.at[r], gsem)

    def s_desc(q, r):
        return pltpu.make_async_copy(rows.at[r], acc.at[didx.at[q]], ssem)

    def step(c, k, warm):
        q, r = k % QN, k % RNG
        qm1, rm1 = (k - 1) % QN, (k - 1) % RNG
        qm2, rm2 = (k - 2) % QN, (k - 2) % RNG
        idx_wait(q)
        if warm >= 2:
            s_desc(qm2, rm2).wait()
        g_desc(q, r).start()
        idx_start(c + 4, (k + 4) % QN)
        if warm >= 1:
            g_desc(qm1, rm1).wait()
            s_desc(qm1, rm1).start(add=True)

    for q in range(4):
        idx_start(jnp.int32(q), q)
    step(jnp.int32(0), 0, 0)
    step(jnp.int32(1), 1, 1)
    for k in range(2, 8):
        step(jnp.int32(k), k, 2)

    def superiter(i, carry):
        for k in range(8):
            step(i * 8 + k, k, 2)
        return carry

    lax.fori_loop(1, cpw // 8, superiter, 0)

    g_desc(7, 3).wait()
    s_desc(7, 3).start(add=True)
    s_desc(6, 2).wait()
    s_desc(7, 3).wait()
    for q in range(4):
        idx_wait(q)
    plsc.subcore_barrier()
    pltpu.sync_copy(acc.at[pl.ds(sid * RPT, RPT)],
                    out_hbm.at[pl.ds(cid * N1 + sid * RPT, RPT)])


BN = 1000
GRID = (N // BN,)


def _row_spec():
    return pl.BlockSpec((BN, H), lambda i: (i, 0))


def _dis_spec():
    return pl.BlockSpec((BN, 16), lambda i: (i, 0))


def _w_spec():
    return pl.BlockSpec((H, H), lambda i: (0, 0))


def _b_spec():
    return pl.BlockSpec((1, H), lambda i: (0, 0))


def _dot(a, b):
    return jnp.dot(a, b, preferred_element_type=f32)


def _stage_b_body(dega_ref, degb_ref, x_ref, dis_ref, xs_ref):
    deg = dega_ref[...][:, :1] + degb_ref[...][:, :1]
    dis = jnp.where(deg > 0.0, 1.0 / jnp.sqrt(jnp.maximum(deg, 1e-12)), 0.0)
    dis_ref[...] = jnp.broadcast_to(dis, dis_ref.shape)
    xs_ref[...] = dis * x_ref[...]


def _stage_c_body(pa_ref, pb_ref, dis_ref, x_ref, w10_ref, w11_ref,
                  u1_ref, acc_ref):
    d = dis_ref[...][:, :1]
    tx1 = -(d * (pa_ref[...] + pb_ref[...]))
    u1_ref[...] = d * tx1
    acc_ref[...] = _dot(x_ref[...], w10_ref[...]) + _dot(tx1, w11_ref[...])


def _stage_d_body(pa_ref, pb_ref, dis_ref, x_ref, acc_ref, w12_ref, b1_ref,
                  h_ref, hs_ref):
    d = dis_ref[...][:, :1]
    tx2 = -2.0 * (d * (pa_ref[...] + pb_ref[...])) - x_ref[...]
    pre = acc_ref[...] + _dot(tx2, w12_ref[...]) + b1_ref[...]
    h = jnp.maximum(pre, 0.0)
    h_ref[...] = h
    hs_ref[...] = d * h


def _stage_e_body(pa_ref, pb_ref, dis_ref, h_ref, w20_ref, w21_ref,
                  u2_ref, acc2_ref):
    d = dis_ref[...][:, :1]
    t1 = -(d * (pa_ref[...] + pb_ref[...]))
    u2_ref[...] = d * t1
    acc2_ref[...] = _dot(h_ref[...], w20_ref[...]) + _dot(t1, w21_ref[...])


def _stage_f_body(pa_ref, pb_ref, dis_ref, h_ref, acc2_ref, w22_ref, b2_ref,
                  wl_ref, bl_ref, o_ref):
    d = dis_ref[...][:, :1]
    t2 = -2.0 * (d * (pa_ref[...] + pb_ref[...])) - h_ref[...]
    hf = h_ref[...] + acc2_ref[...] + _dot(t2, w22_ref[...]) + b2_ref[...]
    logits = _dot(hf, wl_ref[...]) + bl_ref[...]
    lane = lax.broadcasted_iota(i32, logits.shape, 1)
    valid = lane < NCLS
    masked = jnp.where(valid, logits, -1e30)
    m = jnp.max(masked, axis=1, keepdims=True)
    ex = jnp.where(valid, jnp.exp(logits - m), 0.0)
    lse = jnp.log(jnp.sum(ex, axis=1, keepdims=True)) + m
    o_ref[...] = logits - lse


_stage_b = pl.pallas_call(
    _stage_b_body,
    grid=GRID,
    in_specs=[_row_spec(), _row_spec(), _row_spec()],
    out_specs=[_dis_spec(), _row_spec()],
    out_shape=[jax.ShapeDtypeStruct((N, 16), f32),
               jax.ShapeDtypeStruct((N, H), f32)],
)

_stage_c = pl.pallas_call(
    _stage_c_body,
    grid=GRID,
    in_specs=[_row_spec(), _row_spec(), _dis_spec(), _row_spec(),
              _w_spec(), _w_spec()],
    out_specs=[_row_spec(), _row_spec()],
    out_shape=[jax.ShapeDtypeStruct((N, H), f32),
               jax.ShapeDtypeStruct((N, H), f32)],
)

_stage_d = pl.pallas_call(
    _stage_d_body,
    grid=GRID,
    in_specs=[_row_spec(), _row_spec(), _dis_spec(), _row_spec(), _row_spec(),
              _w_spec(), _b_spec()],
    out_specs=[_row_spec(), _row_spec()],
    out_shape=[jax.ShapeDtypeStruct((N, H), f32),
               jax.ShapeDtypeStruct((N, H), f32)],
)

_stage_e = pl.pallas_call(
    _stage_e_body,
    grid=GRID,
    in_specs=[_row_spec(), _row_spec(), _dis_spec(), _row_spec(),
              _w_spec(), _w_spec()],
    out_specs=[_row_spec(), _row_spec()],
    out_shape=[jax.ShapeDtypeStruct((N, H), f32),
               jax.ShapeDtypeStruct((N, H), f32)],
)

_stage_f = pl.pallas_call(
    _stage_f_body,
    grid=GRID,
    in_specs=[_row_spec(), _row_spec(), _dis_spec(), _row_spec(), _row_spec(),
              _w_spec(), _b_spec(), _w_spec(), _b_spec()],
    out_specs=_row_spec(),
    out_shape=jax.ShapeDtypeStruct((N, H), f32),
)


def kernel(x, edge_index, W1, b1, W2, b2, Wl, bl):
    src = jnp.concatenate([edge_index[0].astype(i32), jnp.zeros((E2 - E,), i32)])
    dst = jnp.concatenate([edge_index[1].astype(i32), jnp.zeros((E2 - E,), i32)])
    ones = jnp.ones((CH, H), f32)
    zrow = jnp.zeros((N1, H), f32)

    _edge_prep = _build_edge_prep()
    _prop = _build_prop()
    dstp, degp = _edge_prep(src, dst, zrow, ones)
    dis16, xs = _stage_b(degp[0:N], degp[N1:N1 + N], x)

    p = _prop(xs, src, dstp, zrow)
    u1, acc1 = _stage_c(p[0:N], p[N1:N1 + N], dis16, x, W1[0], W1[1])

    p = _prop(u1, src, dstp, zrow)
    h, hs = _stage_d(p[0:N], p[N1:N1 + N], dis16, x, acc1, W1[2],
                     b1.reshape(1, H))

    p = _prop(hs, src, dstp, zrow)
    u2, acc2 = _stage_e(p[0:N], p[N1:N1 + N], dis16, h, W2[0], W2[1])

    p = _prop(u2, src, dstp, zrow)
    wlp = jnp.zeros((H, H), f32).at[:, :NCLS].set(Wl)
    blp = jnp.zeros((1, H), f32).at[0, :NCLS].set(bl)
    outp = _stage_f(p[0:N], p[N1:N1 + N], dis16, h, acc2, W2[2],
                    b2.reshape(1, H), wlp, blp)

    return outp[:, :NCLS], edge_index
